# baseline (speedup 1.0000x reference)
import jax
import jax.numpy as jnp
from jax.experimental import pallas as pl
from jax.experimental.pallas import tpu as pltpu

_N = 64
_NF = 64
_EF = 16
_MS = 64
_OF = 64
_PASSES = 3
_G = 4


def _mpnn_block(adj_ref, nodes_ref, edges_ref, wmsg_ref, wupd_ref, wout_ref,
                out_ref):
    adj = adj_ref[...]
    nodes = nodes_ref[...]
    edges = edges_ref[...]

    wmsg_n = wmsg_ref[:_NF, :]
    wmsg_e = wmsg_ref[_NF:, :]
    wupd_h = wupd_ref[:_NF, :]
    wupd_m = wupd_ref[_NF:, :]
    wout_h = wout_ref[:_NF, :]
    wout_n = wout_ref[_NF:, :]

    e_proj = jnp.dot(edges.reshape(_G * _N * _N, _EF), wmsg_e,
                     preferred_element_type=jnp.float32)
    e_proj = e_proj.reshape(_G, _N, _N, _MS)

    mask = (jnp.sum(adj, axis=2) != 0).astype(jnp.float32)
    mask3 = mask[:, :, None]
    adj4 = adj[:, :, :, None]

    hidden = nodes
    for _ in range(_PASSES):
        h_proj = jnp.dot(hidden.reshape(_G * _N, _NF), wmsg_n,
                         preferred_element_type=jnp.float32)
        h_proj = h_proj.reshape(_G, 1, _N, _MS)
        msgs = jnp.sum(jnp.tanh(e_proj + h_proj) * adj4, axis=2)
        pre = (jnp.dot(hidden.reshape(_G * _N, _NF), wupd_h,
                       preferred_element_type=jnp.float32)
               + jnp.dot(msgs.reshape(_G * _N, _MS), wupd_m,
                         preferred_element_type=jnp.float32))
        upd = jnp.tanh(pre).reshape(_G, _N, _NF)
        hidden = jnp.where(mask3 != 0, upd, hidden)

    h_sum = jnp.sum(hidden * mask3, axis=1)
    n_sum = jnp.sum(nodes * mask3, axis=1)
    out = (jnp.dot(h_sum, wout_h, preferred_element_type=jnp.float32)
           + jnp.dot(n_sum, wout_n, preferred_element_type=jnp.float32))
    out_ref[...] = out[None]


@jax.jit
def kernel(adjacency, nodes, edges, W_msg, W_upd, W_out):
    b = adjacency.shape[0]
    grid = (b // _G,)
    return pl.pallas_call(
        _mpnn_block,
        grid=grid,
        in_specs=[
            pl.BlockSpec((_G, _N, _N), lambda i: (i, 0, 0)),
            pl.BlockSpec((_G, _N, _NF), lambda i: (i, 0, 0)),
            pl.BlockSpec((_G, _N, _N, _EF), lambda i: (i, 0, 0, 0)),
            pl.BlockSpec((_NF + _EF, _MS), lambda i: (0, 0)),
            pl.BlockSpec((_NF + _MS, _NF), lambda i: (0, 0)),
            pl.BlockSpec((2 * _NF, _OF), lambda i: (0, 0)),
        ],
        out_specs=pl.BlockSpec((1, _G, _OF), lambda i: (i, 0, 0)),
        out_shape=jax.ShapeDtypeStruct((b // _G, _G, _OF), jnp.float32),
        compiler_params=pltpu.CompilerParams(
            dimension_semantics=("arbitrary",),
        ),
    )(adjacency, nodes, edges, W_msg, W_upd, W_out).reshape(b, _OF)


# speedup vs baseline: 1.2125x; 1.2125x over previous
import jax
import jax.numpy as jnp
from jax.experimental import pallas as pl
from jax.experimental.pallas import tpu as pltpu

_N = 64
_NF = 64
_EF = 16
_MS = 64
_OF = 64
_PASSES = 3
_GP = 2


def _mpnn_block(adj_ref, nodes_ref, edges_ref, wmsg_e2_ref, wmsg_n2_ref,
                wupd_h2_ref, wupd_m2_ref, wout_h2_ref, wout_n2_ref, out_ref):
    adj = adj_ref[...].reshape(_GP, 2, _N, _N)
    nodes = nodes_ref[...]
    edges = edges_ref[...]

    a0 = jnp.broadcast_to(adj[:, 0][..., None], (_GP, _N, _N, _MS))
    a1 = jnp.broadcast_to(adj[:, 1][..., None], (_GP, _N, _N, _MS))
    adj_bc = jnp.concatenate([a0, a1], axis=-1)

    deg = jnp.sum(adj, axis=3)
    m0 = jnp.broadcast_to((deg[:, 0] != 0).astype(jnp.float32)[..., None],
                          (_GP, _N, _NF))
    m1 = jnp.broadcast_to((deg[:, 1] != 0).astype(jnp.float32)[..., None],
                          (_GP, _N, _NF))
    mask = jnp.concatenate([m0, m1], axis=-1)

    e_proj = jnp.dot(edges.reshape(_GP * _N * _N, 2 * _EF), wmsg_e2_ref[...],
                     preferred_element_type=jnp.float32)
    e_proj = e_proj.reshape(_GP, _N, _N, 2 * _MS) * adj_bc

    hidden = nodes
    for _ in range(_PASSES):
        h_proj = jnp.dot(hidden.reshape(_GP * _N, 2 * _NF), wmsg_n2_ref[...],
                         preferred_element_type=jnp.float32)
        h_proj = h_proj.reshape(_GP, 1, _N, 2 * _MS)
        msgs = jnp.sum(jnp.tanh(e_proj + adj_bc * h_proj), axis=2)
        pre = (jnp.dot(hidden.reshape(_GP * _N, 2 * _NF), wupd_h2_ref[...],
                       preferred_element_type=jnp.float32)
               + jnp.dot(msgs.reshape(_GP * _N, 2 * _MS), wupd_m2_ref[...],
                         preferred_element_type=jnp.float32))
        upd = jnp.tanh(pre).reshape(_GP, _N, 2 * _NF)
        hidden = hidden + mask * (upd - hidden)

    h_sum = jnp.sum(hidden * mask, axis=1)
    n_sum = jnp.sum(nodes * mask, axis=1)
    out = (jnp.dot(h_sum, wout_h2_ref[...], preferred_element_type=jnp.float32)
           + jnp.dot(n_sum, wout_n2_ref[...],
                     preferred_element_type=jnp.float32))
    out_ref[...] = out[None]


def _blockdiag2(w):
    r, c = w.shape
    z = jnp.zeros((r, c), w.dtype)
    return jnp.concatenate(
        [jnp.concatenate([w, z], axis=1), jnp.concatenate([z, w], axis=1)],
        axis=0)


@jax.jit
def kernel(adjacency, nodes, edges, W_msg, W_upd, W_out):
    b = adjacency.shape[0]
    b2 = b // 2

    nodes_pk = nodes.reshape(b2, 2, _N, _NF).transpose(0, 2, 1, 3) \
        .reshape(b2, _N, 2 * _NF)
    edges_pk = edges.reshape(b2, 2, _N, _N, _EF).transpose(0, 2, 3, 1, 4) \
        .reshape(b2, _N, _N, 2 * _EF)

    wmsg_n2 = _blockdiag2(W_msg[:_NF])
    wmsg_e2 = _blockdiag2(W_msg[_NF:])
    wupd_h2 = _blockdiag2(W_upd[:_NF])
    wupd_m2 = _blockdiag2(W_upd[_NF:])
    wout_h2 = _blockdiag2(W_out[:_NF])
    wout_n2 = _blockdiag2(W_out[_NF:])

    grid = (b2 // _GP,)
    full = lambda i: (0, 0)
    out = pl.pallas_call(
        _mpnn_block,
        grid=grid,
        in_specs=[
            pl.BlockSpec((2 * _GP, _N, _N), lambda i: (i, 0, 0)),
            pl.BlockSpec((_GP, _N, 2 * _NF), lambda i: (i, 0, 0)),
            pl.BlockSpec((_GP, _N, _N, 2 * _EF), lambda i: (i, 0, 0, 0)),
            pl.BlockSpec((2 * _EF, 2 * _MS), full),
            pl.BlockSpec((2 * _NF, 2 * _MS), full),
            pl.BlockSpec((2 * _NF, 2 * _NF), full),
            pl.BlockSpec((2 * _MS, 2 * _NF), full),
            pl.BlockSpec((2 * _NF, 2 * _OF), full),
            pl.BlockSpec((2 * _NF, 2 * _OF), full),
        ],
        out_specs=pl.BlockSpec((1, _GP, 2 * _OF), lambda i: (i, 0, 0)),
        out_shape=jax.ShapeDtypeStruct((b2 // _GP, _GP, 2 * _OF), jnp.float32),
        compiler_params=pltpu.CompilerParams(
            dimension_semantics=("arbitrary",),
        ),
    )(adjacency, nodes_pk, edges_pk, wmsg_e2, wmsg_n2, wupd_h2, wupd_m2,
      wout_h2, wout_n2)
    return out.reshape(b, _OF)
